# baseline (device time: 23657 ns/iter reference)
import jax
import jax.numpy as jnp
from jax import lax
from jax.experimental import pallas as pl
from jax.experimental.pallas import tpu as pltpu

M = 512
HALF = M // 2


def kernel(dy, W):
    m, k = dy.shape
    n = W.shape[0]

    def body(dy_ref, w_ref, out_ref, p_ref, r1_ref, r2_ref,
             s1, v1, s2, v2):
        my_x = lax.axis_index("x")
        my_y = lax.axis_index("y")
        y_nbr = (my_x, 1 - my_y)
        x_nbr = (1 - my_x, my_y)

        barrier_sem = pltpu.get_barrier_semaphore()
        for nbr in (y_nbr, x_nbr):
            pl.semaphore_signal(
                barrier_sem, inc=1,
                device_id=nbr, device_id_type=pl.DeviceIdType.MESH,
            )
        pl.semaphore_wait(barrier_sem, 2)

        row0 = my_x * HALF
        dy_rows = dy_ref[pl.ds(row0, HALF), :]
        p_ref[...] = lax.dot_general(
            dy_rows, w_ref[...],
            dimension_numbers=(((1,), (1,)), ((), ())),
            preferred_element_type=jnp.float32,
        )

        rdma1 = pltpu.make_async_remote_copy(
            src_ref=p_ref, dst_ref=r1_ref,
            send_sem=s1, recv_sem=v1,
            device_id=y_nbr, device_id_type=pl.DeviceIdType.MESH,
        )
        rdma1.start()
        rdma1.wait()

        reduced = p_ref[...] + r1_ref[...]
        out_ref[pl.ds(row0, HALF), :] = reduced
        p_ref[...] = reduced

        rdma2 = pltpu.make_async_remote_copy(
            src_ref=p_ref, dst_ref=r2_ref,
            send_sem=s2, recv_sem=v2,
            device_id=x_nbr, device_id_type=pl.DeviceIdType.MESH,
        )
        rdma2.start()
        rdma2.wait()

        out_ref[pl.ds((1 - my_x) * HALF, HALF), :] = r2_ref[...]

    return pl.pallas_call(
        body,
        out_shape=jax.ShapeDtypeStruct((m, n), jnp.float32),
        in_specs=[
            pl.BlockSpec(memory_space=pltpu.VMEM),
            pl.BlockSpec(memory_space=pltpu.VMEM),
        ],
        out_specs=pl.BlockSpec(memory_space=pltpu.VMEM),
        scratch_shapes=[
            pltpu.VMEM((HALF, n), jnp.float32),
            pltpu.VMEM((HALF, n), jnp.float32),
            pltpu.VMEM((HALF, n), jnp.float32),
            pltpu.SemaphoreType.DMA,
            pltpu.SemaphoreType.DMA,
            pltpu.SemaphoreType.DMA,
            pltpu.SemaphoreType.DMA,
        ],
        compiler_params=pltpu.CompilerParams(collective_id=0),
    )(dy, W)


# device time: 19388 ns/iter; 1.2202x vs baseline; 1.2202x over previous
import jax
import jax.numpy as jnp
from jax import lax
from jax.experimental import pallas as pl
from jax.experimental.pallas import tpu as pltpu

M = 512
HALF = M // 2
C = 4


def kernel(dy, W):
    m, k = dy.shape
    n = W.shape[0]
    cw = n // C

    def body(dy_ref, w_ref, out_ref, p_ref, q_ref, r1_ref, r2_ref,
             s1, v1, s2, v2):
        my_x = lax.axis_index("x")
        my_y = lax.axis_index("y")
        y_nbr = (my_x, 1 - my_y)
        x_nbr = (1 - my_x, my_y)

        def rdma1(c):
            return pltpu.make_async_remote_copy(
                src_ref=p_ref.at[c], dst_ref=r1_ref.at[c],
                send_sem=s1.at[c], recv_sem=v1.at[c],
                device_id=y_nbr, device_id_type=pl.DeviceIdType.MESH,
            )

        def rdma2(c):
            return pltpu.make_async_remote_copy(
                src_ref=q_ref.at[c], dst_ref=r2_ref.at[c],
                send_sem=s2.at[c], recv_sem=v2.at[c],
                device_id=x_nbr, device_id_type=pl.DeviceIdType.MESH,
            )

        barrier_sem = pltpu.get_barrier_semaphore()
        for nbr in (y_nbr, x_nbr):
            pl.semaphore_signal(
                barrier_sem, inc=1,
                device_id=nbr, device_id_type=pl.DeviceIdType.MESH,
            )
        pl.semaphore_wait(barrier_sem, 2)

        row0 = my_x * HALF
        dy_rows = dy_ref[pl.ds(row0, HALF), :]

        for c in range(C):
            p_ref[c] = lax.dot_general(
                dy_rows, w_ref[c * cw:(c + 1) * cw, :],
                dimension_numbers=(((1,), (1,)), ((), ())),
                preferred_element_type=jnp.float32,
            )
            rdma1(c).start()

        for c in range(C):
            rdma1(c).wait_recv()
            red = p_ref[c] + r1_ref[c]
            out_ref[pl.ds(row0, HALF), c * cw:(c + 1) * cw] = red
            q_ref[c] = red
            rdma2(c).start()

        for c in range(C):
            rdma2(c).wait_recv()
            out_ref[pl.ds((1 - my_x) * HALF, HALF), c * cw:(c + 1) * cw] = (
                r2_ref[c]
            )

        for c in range(C):
            rdma1(c).wait_send()
            rdma2(c).wait_send()

    return pl.pallas_call(
        body,
        out_shape=jax.ShapeDtypeStruct((m, n), jnp.float32),
        in_specs=[
            pl.BlockSpec(memory_space=pltpu.VMEM),
            pl.BlockSpec(memory_space=pltpu.VMEM),
        ],
        out_specs=pl.BlockSpec(memory_space=pltpu.VMEM),
        scratch_shapes=[
            pltpu.VMEM((C, HALF, cw), jnp.float32),
            pltpu.VMEM((C, HALF, cw), jnp.float32),
            pltpu.VMEM((C, HALF, cw), jnp.float32),
            pltpu.VMEM((C, HALF, cw), jnp.float32),
            pltpu.SemaphoreType.DMA((C,)),
            pltpu.SemaphoreType.DMA((C,)),
            pltpu.SemaphoreType.DMA((C,)),
            pltpu.SemaphoreType.DMA((C,)),
        ],
        compiler_params=pltpu.CompilerParams(collective_id=0),
    )(dy, W)
